# baseline (speedup 1.0000x reference)
import jax
import jax.numpy as jnp
from jax.experimental import pallas as pl
from jax.experimental.pallas import tpu as pltpu

R, C = 32, 1_000_000
NV = R * C // (8 * 128)
GB = 1250
NBLK = NV // GB


NCHAIN = 10


def _max_body(x_ref, o_ref, acc_ref):
    pid = pl.program_id(0)
    step = GB // NCHAIN
    parts = [
        jnp.max(x_ref[k * step : (k + 1) * step], axis=0) for k in range(NCHAIN)
    ]
    while len(parts) > 1:
        parts = [
            jnp.maximum(parts[i], parts[i + 1]) if i + 1 < len(parts) else parts[i]
            for i in range(0, len(parts), 2)
        ]
    m = parts[0]

    @pl.when(pid == 0)
    def _init():
        acc_ref[...] = m

    @pl.when(pid > 0)
    def _acc():
        acc_ref[...] = jnp.maximum(acc_ref[...], m)

    @pl.when(pid == NBLK - 1)
    def _fin():
        o_ref[0, 0] = jnp.max(acc_ref[...])


def kernel(logits, label_logit):
    x = logits.reshape(NV, 8, 128)
    gmax = pl.pallas_call(
        _max_body,
        grid=(NBLK,),
        in_specs=[pl.BlockSpec((GB, 8, 128), lambda i: (i, 0, 0))],
        out_specs=pl.BlockSpec(memory_space=pltpu.SMEM),
        out_shape=jax.ShapeDtypeStruct((1, 1), jnp.float32),
        scratch_shapes=[pltpu.VMEM((8, 128), jnp.float32)],
    )(x)[0, 0]
    best = jnp.where(gmax >= label_logit, gmax, -jnp.inf)
    return label_logit - best


# speedup vs baseline: 81.5928x; 81.5928x over previous
import jax
import jax.numpy as jnp
from jax.experimental import pallas as pl
from jax.experimental.pallas import tpu as pltpu

R, C = 32, 1_000_000
BW = 32_768
NBLK = (C + BW - 1) // BW
NSL = BW // 128
NCHAIN = 8
_TAIL_FULL = (C - (NBLK - 1) * BW) // 128
_TAIL_REM = (C - (NBLK - 1) * BW) % 128


def _tree(parts):
    while len(parts) > 1:
        parts = [
            jnp.maximum(parts[i], parts[i + 1]) if i + 1 < len(parts) else parts[i]
            for i in range(0, len(parts), 2)
        ]
    return parts[0]


def _reduce_slices(x_ref, ks):
    chains = []
    for c in range(NCHAIN):
        sub = ks[c::NCHAIN]
        if not sub:
            continue
        m = x_ref[:, sub[0] * 128 : sub[0] * 128 + 128]
        for k in sub[1:]:
            m = jnp.maximum(m, x_ref[:, k * 128 : k * 128 + 128])
        chains.append(m)
    return _tree(chains)


def _max_body(x_ref, o_ref, acc_ref):
    pid = pl.program_id(0)

    @pl.when(pid == 0)
    def _init():
        acc_ref[...] = _reduce_slices(x_ref, list(range(NSL)))

    @pl.when((pid > 0) & (pid < NBLK - 1))
    def _main():
        acc_ref[...] = jnp.maximum(acc_ref[...], _reduce_slices(x_ref, list(range(NSL))))

    @pl.when(pid == NBLK - 1)
    def _tail():
        m = _reduce_slices(x_ref, list(range(_TAIL_FULL)))
        part = x_ref[:, _TAIL_FULL * 128 : _TAIL_FULL * 128 + 128]
        lane = jax.lax.broadcasted_iota(jnp.int32, (R, 128), 1)
        m = jnp.maximum(m, jnp.where(lane < _TAIL_REM, part, -jnp.inf))
        o_ref[0, 0] = jnp.max(jnp.maximum(acc_ref[...], m))


def kernel(logits, label_logit):
    gmax = pl.pallas_call(
        _max_body,
        grid=(NBLK,),
        in_specs=[pl.BlockSpec((R, BW), lambda i: (0, i))],
        out_specs=pl.BlockSpec(memory_space=pltpu.SMEM),
        out_shape=jax.ShapeDtypeStruct((1, 1), jnp.float32),
        scratch_shapes=[pltpu.VMEM((R, 128), jnp.float32)],
    )(logits)[0, 0]
    best = jnp.where(gmax >= label_logit, gmax, -jnp.inf)
    return label_logit - best
